# baseline (speedup 1.0000x reference)
import functools

import jax
import jax.numpy as jnp
from jax import lax
from jax.experimental import pallas as pl
from jax.experimental.pallas import tpu as pltpu

B, S, HID = 1, 2048, 2048
H, D, ROPE, NOPE, QLORA, TOPK = 16, 128, 64, 64, 1536, 2048


def _hadamard_transform(x, scale):
    shp = x.shape
    n = shp[-1]
    y = x.reshape(-1, n)
    h = 1
    while h < n:
        y = y.reshape(-1, n // (2 * h), 2, h)
        a = y[:, :, 0, :]
        b = y[:, :, 1, :]
        y = jnp.stack([a + b, a - b], axis=2)
        y = y.reshape(-1, n)
        h *= 2
    return (y * scale).reshape(shp)


def _rotate_activation(x):
    xb = x.astype(jnp.bfloat16)
    return _hadamard_transform(xb, xb.shape[-1] ** (-0.5))


def _apply_rope(x, angles):
    cos = jnp.cos(angles)
    sin = jnp.sin(angles)
    if x.ndim == 4:
        cos = cos[None, :, None, :]
        sin = sin[None, :, None, :]
    else:
        cos = cos[None, :, :]
        sin = sin[None, :, :]
    xr = x[..., 0::2].astype(jnp.float32)
    xi = x[..., 1::2].astype(jnp.float32)
    yr = xr * cos - xi * sin
    yi = xr * sin + xi * cos
    y = jnp.stack([yr, yi], axis=-1).reshape(x.shape)
    return y.astype(x.dtype)


def _layer_norm(x, g, b, eps=1e-5):
    m = jnp.mean(x, axis=-1, keepdims=True)
    v = jnp.var(x, axis=-1, keepdims=True)
    return (x - m) / jnp.sqrt(v + eps) * g + b


BK = 512


def _scores_kernel(kf_ref, qft_ref, w_ref, out_ref):
    h = pl.program_id(1)
    s = lax.dot_general(kf_ref[...], qft_ref[0],
                        (((1,), (0,)), ((), ())),
                        preferred_element_type=jnp.float32)
    s = jnp.maximum(s, 0.0) * w_ref[0, 0][None, :]

    @pl.when(h == 0)
    def _():
        out_ref[...] = s

    @pl.when(h > 0)
    def _():
        out_ref[...] += s


def _scores_T(kf, qf_t, w_t):
    return pl.pallas_call(
        _scores_kernel,
        grid=(S // BK, H),
        in_specs=[
            pl.BlockSpec((BK, D), lambda i, h: (i, 0)),
            pl.BlockSpec((1, D, S), lambda i, h: (h, 0, 0)),
            pl.BlockSpec((1, 1, S), lambda i, h: (h, 0, 0)),
        ],
        out_specs=pl.BlockSpec((BK, S), lambda i, h: (i, 0)),
        out_shape=jax.ShapeDtypeStruct((S, S), jnp.float32),
    )(kf, qf_t, w_t)


def kernel(x, q_resid, freqs_cis, Wq_b, Wk, k_norm_weight, k_norm_bias, Wweights):
    softmax_scale = D ** (-0.5)
    q = (q_resid @ Wq_b.T).reshape(B, S, H, D)
    q_nope, q_pe = q[..., :NOPE], q[..., NOPE:]
    k = _layer_norm(x @ Wk.T, k_norm_weight, k_norm_bias)
    k_nope, k_pe = k[..., :NOPE], k[..., NOPE:]
    q_pe = _apply_rope(q_pe, freqs_cis)
    k_pe = _apply_rope(k_pe, freqs_cis)
    q = jnp.concatenate([q_nope, q_pe], axis=-1)
    k = jnp.concatenate([k_nope, k_pe], axis=-1)
    q = _rotate_activation(q)
    k = _rotate_activation(k)
    weights = (x @ Wweights.T).astype(jnp.float32) * (H ** (-0.5)) * softmax_scale

    kf = k[0]
    qf_t = jnp.transpose(q[0], (1, 2, 0))
    w_t = jnp.transpose(weights[0], (1, 0))[:, None, :]

    sT = _scores_T(kf, qf_t, w_t)
    scores = jnp.transpose(sT)[None]
    _, topk_indices = lax.top_k(scores, TOPK)
    return topk_indices
